# Initial kernel scaffold; baseline (speedup 1.0000x reference)
import functools

import jax
import jax.numpy as jnp
from jax import lax
from jax.experimental import pallas as pl
from jax.experimental.pallas import tpu as pltpu
from jax.experimental.pallas import tpu_sc as plsc

N = 10000
E = 320000
D = 128
DE = 4
DEP = 16

NC = 2
NS = 16
NW = NC * NS
CHUNK = 128
CPT = (E + NW * CHUNK - 1) // (NW * CHUNK)
E_PAD = NW * CPT * CHUNK
N_ACC = 10240
RPT = N_ACC // NS


def _sc_segsum(x, src2d, dst2d, ea2d):
    mesh = plsc.VectorSubcoreMesh(core_axis_name="c", subcore_axis_name="s")

    @functools.partial(
        pl.kernel,
        out_type=[
            jax.ShapeDtypeStruct((NC, N_ACC, D), jnp.float32),
            jax.ShapeDtypeStruct((NC, N_ACC, DEP), jnp.float32),
        ],
        mesh=mesh,
        scratch_types=[
            pltpu.VMEM((CPT, CHUNK), jnp.int32),
            pltpu.VMEM((CPT, CHUNK), jnp.int32),
            pltpu.VMEM((CHUNK, D), jnp.float32),
            pltpu.VMEM((CHUNK, DEP), jnp.float32),
            pltpu.VMEM_SHARED((N_ACC, D), jnp.float32),
            pltpu.VMEM_SHARED((N_ACC, DEP), jnp.float32),
            pltpu.SemaphoreType.DMA,
        ],
    )
    def k(x_hbm, src_hbm, dst_hbm, ea_hbm, outx_hbm, oute_hbm,
          src_v, dst_v, rows_v, ea_v, accx_sh, acce_sh, sem):
        cid = lax.axis_index("c")
        sid = lax.axis_index("s")
        wid = cid * NS + sid

        zero16 = jnp.zeros((16,), jnp.float32)

        def zx(i, _):
            rows_v[i // (D // 16), pl.ds((i % (D // 16)) * 16, 16)] = zero16
            return 0
        lax.fori_loop(0, CHUNK * (D // 16), zx, 0)

        def ze(i, _):
            ea_v[i, pl.ds(0, 16)] = zero16
            return 0
        lax.fori_loop(0, CHUNK, ze, 0)

        def zcopy(i, _):
            r = sid * RPT + i * CHUNK
            pltpu.sync_copy(rows_v, accx_sh.at[pl.ds(r, CHUNK)])
            pltpu.sync_copy(ea_v, acce_sh.at[pl.ds(r, CHUNK)])
            return 0
        lax.fori_loop(0, RPT // CHUNK, zcopy, 0)

        pltpu.sync_copy(src_hbm.at[wid], src_v)
        pltpu.sync_copy(dst_hbm.at[wid], dst_v)
        plsc.subcore_barrier()

        def body(j, _):
            pltpu.async_copy(x_hbm.at[src_v.at[j]], rows_v, sem).wait()
            pltpu.sync_copy(ea_hbm.at[pl.ds((wid * CPT + j) * CHUNK, CHUNK)],
                            ea_v)
            pltpu.sync_copy(rows_v, accx_sh.at[dst_v.at[j]], add=True)
            pltpu.sync_copy(ea_v, acce_sh.at[dst_v.at[j]], add=True)
            return 0
        lax.fori_loop(0, CPT, body, 0)

        plsc.subcore_barrier()

        r0 = sid * RPT
        pltpu.sync_copy(accx_sh.at[pl.ds(r0, RPT)],
                        outx_hbm.at[cid].at[pl.ds(r0, RPT)])
        pltpu.sync_copy(acce_sh.at[pl.ds(r0, RPT)],
                        oute_hbm.at[cid].at[pl.ds(r0, RPT)])

    return k(x, src2d, dst2d, ea2d)


def _tc_update(px, pe, x, wm1, wm2, ws, b2d):
    R = 1000

    def body(px_ref, pe_ref, x_ref, wm1_ref, wm2_ref, ws_ref, b_ref, o_ref):
        sx = px_ref[0] + px_ref[1]
        se = pe_ref[0] + pe_ref[1]
        acc = jnp.dot(sx, wm1_ref[...], preferred_element_type=jnp.float32)
        acc += jnp.dot(se, wm2_ref[...], preferred_element_type=jnp.float32)
        acc += jnp.dot(x_ref[...], ws_ref[...],
                       preferred_element_type=jnp.float32)
        acc += b_ref[...]
        o_ref[...] = acc * jax.nn.sigmoid(acc)

    return pl.pallas_call(
        body,
        grid=(N // R,),
        in_specs=[
            pl.BlockSpec((NC, R, D), lambda i: (0, i, 0)),
            pl.BlockSpec((NC, R, DEP), lambda i: (0, i, 0)),
            pl.BlockSpec((R, D), lambda i: (i, 0)),
            pl.BlockSpec((D, D), lambda i: (0, 0)),
            pl.BlockSpec((DEP, D), lambda i: (0, 0)),
            pl.BlockSpec((D, D), lambda i: (0, 0)),
            pl.BlockSpec((1, D), lambda i: (0, 0)),
        ],
        out_specs=pl.BlockSpec((R, D), lambda i: (i, 0)),
        out_shape=jax.ShapeDtypeStruct((N, D), jnp.float32),
    )(px, pe, x, wm1, wm2, ws, b2d)


def kernel(x, edge_index, edge_attr, W_msg, W_self, b):
    src = edge_index[0]
    dst = edge_index[1]

    pad = E_PAD - E
    src_p = jnp.concatenate([src, jnp.zeros((pad,), jnp.int32)])
    dst_p = jnp.concatenate([dst, jnp.full((pad,), N, jnp.int32)])
    src2d = src_p.reshape(NW, CPT, CHUNK)
    dst2d = dst_p.reshape(NW, CPT, CHUNK)
    ea2d = jnp.pad(edge_attr, ((0, pad), (0, DEP - DE)))

    px, pe = _sc_segsum(x, src2d, dst2d, ea2d)

    wm1 = W_msg[:D]
    wm2 = jnp.pad(W_msg[D:], ((0, DEP - DE), (0, 0)))
    b2d = b.reshape(1, D)
    return _tc_update(px[:, :N], pe[:, :N], x, wm1, wm2, W_self, b2d)


# speedup vs baseline: 2.8099x; 2.8099x over previous
import functools

import jax
import jax.numpy as jnp
from jax import lax
from jax.experimental import pallas as pl
from jax.experimental.pallas import tpu as pltpu
from jax.experimental.pallas import tpu_sc as plsc

N = 10000
E = 320000
D = 128
DH = 64
DE = 4
DEP = 16

NC = 2
NS = 16
NW = NC * NS
CHUNK = 128
CPT = (E + NW * CHUNK - 1) // (NW * CHUNK)
E_PAD = NW * CPT * CHUNK
N_ACC = 10240
RPT = N_ACC // NS


def _sc_segsum(x2, src2d, dst2d, ea2d):
    mesh = plsc.VectorSubcoreMesh(core_axis_name="c", subcore_axis_name="s")

    @functools.partial(
        pl.kernel,
        out_type=[
            jax.ShapeDtypeStruct((NC * N_ACC, DH), jnp.float32),
            jax.ShapeDtypeStruct((NC * N_ACC, DH), jnp.float32),
            jax.ShapeDtypeStruct((NC * N_ACC, DEP), jnp.float32),
        ],
        mesh=mesh,
        compiler_params=pltpu.CompilerParams(use_tc_tiling_on_sc=False),
        scratch_types=[
            pltpu.VMEM((1, CHUNK), jnp.int32),
            pltpu.VMEM((1, CHUNK), jnp.int32),
            pltpu.VMEM((1, CHUNK), jnp.int32),
            pltpu.VMEM((1, CHUNK), jnp.int32),
            pltpu.VMEM((CHUNK, DH), jnp.float32),
            pltpu.VMEM((CHUNK, DH), jnp.float32),
            pltpu.VMEM((CHUNK, DEP), jnp.float32),
            pltpu.VMEM_SHARED((N_ACC, DH), jnp.float32),
            pltpu.VMEM_SHARED((N_ACC, DH), jnp.float32),
            pltpu.VMEM_SHARED((N_ACC, DEP), jnp.float32),
            pltpu.SemaphoreType.DMA,
            pltpu.SemaphoreType.DMA,
        ],
    )
    def k(x_hbm, src_hbm, dst_hbm, ea_hbm, outl_hbm, outr_hbm, oute_hbm,
          src_c, dst_c, srcl_c, srcr_c, rowsl_v, rowsr_v, ea_v,
          accl_sh, accr_sh, acce_sh, seml, semr):
        cid = lax.axis_index("c")
        sid = lax.axis_index("s")
        wid = cid * NS + sid

        zero16 = jnp.zeros((16,), jnp.float32)

        def zx(i, _):
            rowsl_v[i // (DH // 16), pl.ds((i % (DH // 16)) * 16, 16)] = zero16
            return 0
        lax.fori_loop(0, CHUNK * (DH // 16), zx, 0)

        def ze(i, _):
            ea_v[i, pl.ds(0, 16)] = zero16
            return 0
        lax.fori_loop(0, CHUNK, ze, 0)

        @pl.loop(0, RPT // CHUNK)
        def zcopy(i):
            r = sid * RPT + i * CHUNK
            pltpu.sync_copy(rowsl_v, accl_sh.at[pl.ds(r, CHUNK)])
            pltpu.sync_copy(rowsl_v, accr_sh.at[pl.ds(r, CHUNK)])
            pltpu.sync_copy(ea_v, acce_sh.at[pl.ds(r, CHUNK)])

        plsc.subcore_barrier()

        @pl.loop(0, CPT)
        def body(j):
            c = wid * CPT + j
            pltpu.sync_copy(src_hbm.at[pl.ds(c, 1)], src_c)
            pltpu.sync_copy(dst_hbm.at[pl.ds(c, 1)], dst_c)
            for t in range(CHUNK // 16):
                s = src_c[0, pl.ds(t * 16, 16)]
                s2 = s + s
                srcl_c[0, pl.ds(t * 16, 16)] = s2
                srcr_c[0, pl.ds(t * 16, 16)] = s2 + 1
            gl = pltpu.async_copy(x_hbm.at[srcl_c.at[0]], rowsl_v, seml)
            gr = pltpu.async_copy(x_hbm.at[srcr_c.at[0]], rowsr_v, semr)
            pltpu.sync_copy(ea_hbm.at[pl.ds(c * CHUNK, CHUNK)], ea_v)
            pltpu.sync_copy(ea_v, acce_sh.at[dst_c.at[0]], add=True)
            gl.wait()
            pltpu.sync_copy(rowsl_v, accl_sh.at[dst_c.at[0]], add=True)
            gr.wait()
            pltpu.sync_copy(rowsr_v, accr_sh.at[dst_c.at[0]], add=True)

        plsc.subcore_barrier()

        r0 = sid * RPT
        o0 = cid * N_ACC + r0
        pltpu.sync_copy(accl_sh.at[pl.ds(r0, RPT)], outl_hbm.at[pl.ds(o0, RPT)])
        pltpu.sync_copy(accr_sh.at[pl.ds(r0, RPT)], outr_hbm.at[pl.ds(o0, RPT)])
        pltpu.sync_copy(acce_sh.at[pl.ds(r0, RPT)], oute_hbm.at[pl.ds(o0, RPT)])

    return k(x2, src2d, dst2d, ea2d)


def _tc_update(pxl, pxr, pe, x, wm1a, wm1b, wm2, ws, b2d):
    R = 1000

    def body(pxl_ref, pxr_ref, pe_ref, x_ref, wm1a_ref, wm1b_ref, wm2_ref,
             ws_ref, b_ref, o_ref):
        sxl = pxl_ref[0] + pxl_ref[1]
        sxr = pxr_ref[0] + pxr_ref[1]
        se = pe_ref[0] + pe_ref[1]
        acc = jnp.dot(sxl, wm1a_ref[...], preferred_element_type=jnp.float32)
        acc += jnp.dot(sxr, wm1b_ref[...], preferred_element_type=jnp.float32)
        acc += jnp.dot(se, wm2_ref[...], preferred_element_type=jnp.float32)
        acc += jnp.dot(x_ref[...], ws_ref[...],
                       preferred_element_type=jnp.float32)
        acc += b_ref[...]
        o_ref[...] = acc * jax.nn.sigmoid(acc)

    return pl.pallas_call(
        body,
        grid=(N // R,),
        in_specs=[
            pl.BlockSpec((NC, R, DH), lambda i: (0, i, 0)),
            pl.BlockSpec((NC, R, DH), lambda i: (0, i, 0)),
            pl.BlockSpec((NC, R, DEP), lambda i: (0, i, 0)),
            pl.BlockSpec((R, D), lambda i: (i, 0)),
            pl.BlockSpec((DH, D), lambda i: (0, 0)),
            pl.BlockSpec((DH, D), lambda i: (0, 0)),
            pl.BlockSpec((DEP, D), lambda i: (0, 0)),
            pl.BlockSpec((D, D), lambda i: (0, 0)),
            pl.BlockSpec((1, D), lambda i: (0, 0)),
        ],
        out_specs=pl.BlockSpec((R, D), lambda i: (i, 0)),
        out_shape=jax.ShapeDtypeStruct((N, D), jnp.float32),
    )(pxl, pxr, pe, x, wm1a, wm1b, wm2, ws, b2d)


def kernel(x, edge_index, edge_attr, W_msg, W_self, b):
    src = edge_index[0]
    dst = edge_index[1]

    pad = E_PAD - E
    src_p = jnp.concatenate([src, jnp.zeros((pad,), jnp.int32)])
    dst_p = jnp.concatenate([dst, jnp.full((pad,), N, jnp.int32)])
    src2d = src_p.reshape(NW * CPT, CHUNK)
    dst2d = dst_p.reshape(NW * CPT, CHUNK)
    ea2d = jnp.pad(edge_attr, ((0, pad), (0, DEP - DE)))
    x2 = x.reshape(2 * N, DH)

    pxl, pxr, pe = _sc_segsum(x2, src2d, dst2d, ea2d)
    pxl = pxl.reshape(NC, N_ACC, DH)[:, :N]
    pxr = pxr.reshape(NC, N_ACC, DH)[:, :N]
    pe = pe.reshape(NC, N_ACC, DEP)[:, :N]

    wm1a = W_msg[:DH]
    wm1b = W_msg[DH:D]
    wm2 = jnp.pad(W_msg[D:], ((0, DEP - DE), (0, 0)))
    b2d = b.reshape(1, D)
    return _tc_update(pxl, pxr, pe, x, wm1a, wm1b, wm2, W_self, b2d)
